# baseline (speedup 1.0000x reference)
import functools
import math

import jax
import jax.numpy as jnp
from jax import lax
from jax.experimental import pallas as pl
from jax.experimental.pallas import tpu as pltpu
from jax.experimental.pallas import tpu_sc as plsc

_MODEL_DIM = 64
_VOCAB = 1000000
_BATCH = 16384
_SCALE = math.sqrt(_MODEL_DIM)

_info = plsc.get_sparse_core_info()
_NC = _info.num_cores
_NS = _info.num_subcores
_L = _info.num_lanes
_NW = _NC * _NS
_TCOL = 128
_NCOLS = (_VOCAB + _TCOL - 1) // _TCOL
_COLS_PER_W = (_NCOLS + _NW - 1) // _NW
_BINCAP = 16
_HITCAP = 672
_XCHUNK = 2048
_YROWS = _BATCH + 64

_mesh = plsc.VectorSubcoreMesh(core_axis_name="c", subcore_axis_name="s")


@functools.partial(
    pl.kernel,
    mesh=_mesh,
    compiler_params=pltpu.CompilerParams(needs_layout_passes=False),
    out_type=jax.ShapeDtypeStruct((_YROWS, _TCOL), jnp.float32),
    scratch_types=[
        pltpu.VMEM((2, _XCHUNK), jnp.int32),
        pltpu.VMEM((2, _MODEL_DIM, _TCOL), jnp.float32),
        pltpu.VMEM((_COLS_PER_W * _BINCAP,), jnp.int32),
        pltpu.VMEM((_HITCAP, _TCOL), jnp.float32),
        pltpu.VMEM((_HITCAP,), jnp.int32),
        pltpu.SMEM((_COLS_PER_W + 8,), jnp.int32),
        pltpu.SemaphoreType.DMA,
        pltpu.SemaphoreType.DMA,
    ],
)
def _emb_sweep(x_hbm, wt_hbm, y_hbm, xc_v, tb, bins_v, rows_v, pos_v,
               cnt_s, sem, sem2):
    wid = lax.axis_index("s") * _NC + lax.axis_index("c")
    col_lo = wid * _NCOLS // _NW
    col_hi = (wid + 1) * _NCOLS // _NW
    lane = lax.iota(jnp.int32, _L)
    lane0 = lane == 0

    def full(v):
        return jnp.full((_L,), v, jnp.int32)

    def zero_cnt(i, c):
        cnt_s[i] = 0
        return c

    lax.fori_loop(0, _COLS_PER_W + 8, zero_cnt, 0)

    def fill_dump(i, c):
        pos_v[pl.ds(i * _L, _L)] = full(_BATCH + wid) + lane * 0
        return c

    lax.fori_loop(0, _HITCAP // _L, fill_dump, 0)

    first_x = pltpu.async_copy(x_hbm.at[pl.ds(0, _XCHUNK)], xc_v.at[0], sem)
    first_x.wait()

    def scan_chunk(ch, carry):
        nxt = jnp.minimum(ch + 1, _BATCH // _XCHUNK - 1)
        cp = pltpu.async_copy(
            x_hbm.at[pl.ds(nxt * _XCHUNK, _XCHUNK)], xc_v.at[(ch + 1) % 2],
            sem)

        def scan_vec(t, carry2):
            v = xc_v[ch % 2, pl.ds(t * _L, _L)]
            c = lax.shift_right_logical(v, 7)
            mask = (c >= col_lo) & (c < col_hi)

            def has_hits(state):
                m, _ = state
                return plsc.all_reduce_population_count(m)[0] > 0

            def take_hit(state):
                m, _ = state
                l = plsc.all_reduce_ffs(m)[0]
                lv = full(l)
                v_l = jnp.sum(jnp.where(lane == lv, v, 0))
                cl = lax.shift_right_logical(v_l, 7) - col_lo
                m_l = jnp.bitwise_and(v_l, _TCOL - 1)
                pos_l = ch * _XCHUNK + t * _L + l
                n = cnt_s[cl]
                cnt_s[cl] = n + 1
                slot = cl * _BINCAP + jnp.minimum(n, _BINCAP - 1)
                packed = jnp.left_shift(m_l, 16) + pos_l
                plsc.store_scatter(bins_v, [full(slot)], full(packed),
                                   mask=lane0)
                return m & (lane != l), 0

            lax.while_loop(has_hits, take_hit, (mask, 0))
            return carry2

        lax.fori_loop(0, _XCHUNK // _L, scan_vec, 0)
        cp.wait()
        return carry

    lax.fori_loop(0, _BATCH // _XCHUNK, scan_chunk, 0)

    ncols = col_hi - col_lo
    first_c = pltpu.async_copy(
        wt_hbm.at[:, pl.ds(pl.multiple_of(col_lo * _TCOL, _TCOL), _TCOL)],
        tb.at[0], sem)
    first_c.wait()

    def sweep(cl, hw):
        nxt = col_lo + jnp.minimum(cl + 1, ncols - 1)
        cp = pltpu.async_copy(
            wt_hbm.at[:, pl.ds(pl.multiple_of(nxt * _TCOL, _TCOL), _TCOL)],
            tb.at[(cl + 1) % 2], sem)
        binvec = bins_v[pl.ds(cl * _BINCAP, _L)]
        n = jnp.where(cl < ncols, jnp.minimum(cnt_s[cl], _BINCAP), 0)
        slotv = full(cl % 2)

        def extract(h, hw2):
            e = jnp.sum(jnp.where(lane == full(h), binvec, 0))
            m = lax.shift_right_logical(e, 16)
            pos = jnp.bitwise_and(e, 0xFFFF)
            hw_c = jnp.minimum(hw2, _HITCAP - 1)
            plsc.store_scatter(pos_v, [full(hw_c)], full(pos), mask=lane0)
            mv = full(m)
            for f16 in range(_MODEL_DIM // _L):
                fvec = lane + f16 * _L
                vals = plsc.load_gather(tb, [slotv, fvec, mv]) * _SCALE
                rows_v[hw_c, pl.ds(f16 * _L, _L)] = vals
            return hw2 + 1

        hw = lax.fori_loop(0, n, extract, hw)
        cp.wait()
        return hw

    lax.fori_loop(0, _COLS_PER_W, sweep, 0)

    pltpu.async_copy(rows_v, y_hbm.at[pos_v], sem2).wait()


def kernel(x, W):
    y = _emb_sweep(x, W.T)
    return y[:_BATCH, :_MODEL_DIM]
